# Initial kernel scaffold; baseline (speedup 1.0000x reference)
import jax
import jax.numpy as jnp
from jax.experimental import pallas as pl


def kernel(x, edge_index, batch, W1_0, b1_0, W2_0, b2_0, W1_1, b1_1, W2_1, b2_1, W1_2, b1_2, W2_2, b2_2, W_out, b_out):
    raise NotImplementedError("write your pallas kernel here")


# speedup vs baseline: 5.8245x; 5.8245x over previous
import functools

import jax
import jax.numpy as jnp
from jax import lax
from jax.experimental import pallas as pl
from jax.experimental.pallas import tpu as pltpu
from jax.experimental.pallas import tpu_sc as plsc

N, E, D, H, O, G = 10000, 320000, 128, 128, 64, 64
NC, NS = 2, 16
NW = NC * NS
CHUNK = 128
CHUNKS = E // CHUNK
ROWS_PER_SUB = 632
NPAD = NS * ROWS_PER_SUB

_mesh = plsc.VectorSubcoreMesh(core_axis_name="c", subcore_axis_name="s")


@functools.partial(
    pl.kernel,
    out_type=jax.ShapeDtypeStruct((NC, NPAD, D), jnp.float32),
    mesh=_mesh,
    scratch_types=[
        pltpu.VMEM((CHUNK,), jnp.int32),
        pltpu.VMEM((CHUNK,), jnp.int32),
        pltpu.VMEM((CHUNK, D), jnp.float32),
        pltpu.VMEM_SHARED((NPAD, D), jnp.float32),
        pltpu.SemaphoreType.DMA,
    ],
)
def _sc_edge_aggregate(h_hbm, src_hbm, dst_hbm, out_hbm, src_v, dst_v, rows_v, agg_sh, sem):
    c = lax.axis_index("c")
    s = lax.axis_index("s")
    w = s * NC + c

    @pl.loop(0, CHUNK)
    def _(r):
        for c16 in range(D // 16):
            rows_v[r, pl.ds(c16 * 16, 16)] = jnp.zeros((16,), jnp.float32)

    row0 = s * ROWS_PER_SUB
    off = 0
    while off < ROWS_PER_SUB:
        step = min(CHUNK, ROWS_PER_SUB - off)
        pltpu.sync_copy(rows_v.at[pl.ds(0, step)],
                        agg_sh.at[pl.ds(row0 + off, step)])
        off += step
    plsc.subcore_barrier()

    per = CHUNKS // NW
    extra = CHUNKS - NW * per
    base = w * per + jnp.minimum(w, extra)
    cnt = per + (w < extra).astype(jnp.int32)

    @pl.loop(0, cnt)
    def _(j):
        r = base + j
        pltpu.sync_copy(src_hbm.at[r], src_v)
        pltpu.sync_copy(dst_hbm.at[r], dst_v)
        pltpu.async_copy(h_hbm.at[src_v], rows_v, sem).wait()
        pltpu.sync_copy(rows_v, agg_sh.at[dst_v], add=True)

    plsc.subcore_barrier()
    pltpu.sync_copy(agg_sh.at[pl.ds(row0, ROWS_PER_SUB)],
                    out_hbm.at[c, pl.ds(row0, ROWS_PER_SUB)])


BLK = 2000


def _mlp_body(h_ref, a_ref, w1_ref, b1_ref, w2_ref, b2_ref, o_ref):
    hh = h_ref[...] + a_ref[0] + a_ref[1]
    t = jnp.maximum(jnp.dot(hh, w1_ref[...], preferred_element_type=jnp.float32)
                    + b1_ref[...], 0.0)
    o = jnp.dot(t, w2_ref[...], preferred_element_type=jnp.float32) + b2_ref[...]
    o_ref[...] = jnp.maximum(o, 0.0)


_tc_mlp = pl.pallas_call(
    _mlp_body,
    grid=(N // BLK,),
    in_specs=[
        pl.BlockSpec((BLK, D), lambda i: (i, 0)),
        pl.BlockSpec((NC, BLK, D), lambda i: (0, i, 0)),
        pl.BlockSpec((D, H), lambda i: (0, 0)),
        pl.BlockSpec((1, H), lambda i: (0, 0)),
        pl.BlockSpec((H, H), lambda i: (0, 0)),
        pl.BlockSpec((1, H), lambda i: (0, 0)),
    ],
    out_specs=pl.BlockSpec((BLK, H), lambda i: (i, 0)),
    out_shape=jax.ShapeDtypeStruct((N, H), jnp.float32),
)


def _final_body(h_ref, a_ref, w1_ref, b1_ref, w2_ref, b2_ref,
                batch_ref, wo_ref, bo_ref, o_ref, acc_ref):
    i = pl.program_id(0)
    hh = h_ref[...] + a_ref[0] + a_ref[1]
    t = jnp.maximum(jnp.dot(hh, w1_ref[...], preferred_element_type=jnp.float32)
                    + b1_ref[...], 0.0)
    t = jnp.maximum(jnp.dot(t, w2_ref[...], preferred_element_type=jnp.float32)
                    + b2_ref[...], 0.0)
    mask = (batch_ref[0] == lax.broadcasted_iota(jnp.int32, (G, BLK), 0)
            ).astype(jnp.float32)
    part = jnp.dot(mask, t, preferred_element_type=jnp.float32)

    @pl.when(i == 0)
    def _():
        acc_ref[...] = part

    @pl.when(i > 0)
    def _():
        acc_ref[...] += part

    @pl.when(i == N // BLK - 1)
    def _():
        o_ref[...] = (jnp.dot(acc_ref[...], wo_ref[...],
                              preferred_element_type=jnp.float32) + bo_ref[...])


_tc_final = pl.pallas_call(
    _final_body,
    grid=(N // BLK,),
    in_specs=[
        pl.BlockSpec((BLK, D), lambda i: (i, 0)),
        pl.BlockSpec((NC, BLK, D), lambda i: (0, i, 0)),
        pl.BlockSpec((D, H), lambda i: (0, 0)),
        pl.BlockSpec((1, H), lambda i: (0, 0)),
        pl.BlockSpec((H, H), lambda i: (0, 0)),
        pl.BlockSpec((1, H), lambda i: (0, 0)),
        pl.BlockSpec((1, 1, BLK), lambda i: (i, 0, 0)),
        pl.BlockSpec((H, O), lambda i: (0, 0)),
        pl.BlockSpec((1, O), lambda i: (0, 0)),
    ],
    out_specs=pl.BlockSpec((G, O), lambda i: (0, 0)),
    out_shape=jax.ShapeDtypeStruct((G, O), jnp.float32),
    scratch_shapes=[pltpu.VMEM((G, H), jnp.float32)],
)


def kernel(x, edge_index, batch, W1_0, b1_0, W2_0, b2_0, W1_1, b1_1, W2_1, b2_1,
           W1_2, b1_2, W2_2, b2_2, W_out, b_out):
    src2 = edge_index[0].reshape(CHUNKS, CHUNK)
    dst2 = edge_index[1].reshape(CHUNKS, CHUNK)
    batch3 = batch.reshape(N // BLK, 1, BLK)
    Ws = [(W1_0, b1_0, W2_0, b2_0), (W1_1, b1_1, W2_1, b2_1), (W1_2, b1_2, W2_2, b2_2)]

    h = x
    for i in range(3):
        W1, b1, W2, b2 = Ws[i]
        aggs = _sc_edge_aggregate(h, src2, dst2)
        b1r = b1.reshape(1, H)
        b2r = b2.reshape(1, H)
        if i < 2:
            h = _tc_mlp(h, aggs, W1, b1r, W2, b2r)
        else:
            out = _tc_final(h, aggs, W1, b1r, W2, b2r, batch3,
                            W_out, b_out.reshape(1, O))
    return out
